# baseline (speedup 1.0000x reference)
import functools

import jax
import jax.numpy as jnp
from jax import lax
from jax.experimental import pallas as pl
from jax.experimental.pallas import tpu as pltpu
from jax.experimental.pallas import tpu_sc as plsc

NC = 2
NS = 16
NW = NC * NS
CHUNK = 128


@functools.partial(jax.jit, static_argnames=("n_chunks", "d"))
def _gather(idx3d, weight, n_chunks, d):
    n = NW * n_chunks * CHUNK
    mesh = plsc.VectorSubcoreMesh(core_axis_name="c", subcore_axis_name="s")

    @functools.partial(
        pl.kernel,
        mesh=mesh,
        compiler_params=pltpu.CompilerParams(use_tc_tiling_on_sc=False),
        out_type=jax.ShapeDtypeStruct((n, d), jnp.float32),
        scratch_types=[
            pltpu.VMEM((n_chunks, CHUNK), jnp.int32),
            pltpu.VMEM((CHUNK, d), jnp.float32),
            pltpu.SemaphoreType.DMA,
        ],
    )
    def k(idx_hbm, tbl_hbm, out_hbm, idx_v, rows_v, gsem):
        wid = lax.axis_index("s") * NC + lax.axis_index("c")
        pltpu.sync_copy(idx_hbm.at[wid], idx_v)
        out_base = wid * n_chunks * CHUNK

        def body(j, carry):
            pltpu.async_copy(tbl_hbm.at[idx_v.at[j]], rows_v, gsem).wait()
            pltpu.sync_copy(
                rows_v, out_hbm.at[pl.ds(out_base + j * CHUNK, CHUNK)]
            )
            return carry

        lax.fori_loop(0, n_chunks, body, 0)

    return k(idx3d, weight)


def kernel(idx, weight):
    b, h = idx.shape
    v, d = weight.shape
    n = b * h
    n_chunks = n // (NW * CHUNK)
    idx3d = idx.reshape(NW, n_chunks, CHUNK)
    out = _gather(idx3d, weight, n_chunks, d)
    return out.reshape(b, h, d)


# speedup vs baseline: 1.0463x; 1.0463x over previous
import functools

import jax
import jax.numpy as jnp
from jax import lax
from jax.experimental import pallas as pl
from jax.experimental.pallas import tpu as pltpu
from jax.experimental.pallas import tpu_sc as plsc

NC = 2
NS = 16
NW = NC * NS
CHUNK = 128
NBUF = 8


@functools.partial(jax.jit, static_argnames=("n_chunks", "d"))
def _gather(idx3d, weight, n_chunks, d):
    n = NW * n_chunks * CHUNK
    mesh = plsc.VectorSubcoreMesh(core_axis_name="c", subcore_axis_name="s")

    @functools.partial(
        pl.kernel,
        mesh=mesh,
        compiler_params=pltpu.CompilerParams(use_tc_tiling_on_sc=False),
        out_type=jax.ShapeDtypeStruct((n, d), jnp.float32),
        scratch_types=[
            pltpu.VMEM((n_chunks, CHUNK), jnp.int32),
            pltpu.VMEM((NBUF, CHUNK, d), jnp.float32),
            pltpu.SemaphoreType.DMA,
            pltpu.SemaphoreType.DMA,
        ],
    )
    def k(idx_hbm, tbl_hbm, out_hbm, idx_v, rows_v, gsem, ssem):
        wid = lax.axis_index("s") * NC + lax.axis_index("c")
        pltpu.sync_copy(idx_hbm.at[wid], idx_v)
        out_base = wid * n_chunks * CHUNK

        def gather(g, b):
            pltpu.async_copy(tbl_hbm.at[idx_v.at[g]], rows_v.at[b], gsem)

        for b in range(NBUF):
            gather(b, b)

        def body(j, carry):
            b = lax.rem(j, NBUF)
            pltpu.make_async_copy(
                tbl_hbm.at[idx_v.at[j]], rows_v.at[b], gsem
            ).wait()
            pltpu.async_copy(
                rows_v.at[b], out_hbm.at[pl.ds(out_base + j * CHUNK, CHUNK)],
                ssem,
            )

            @pl.when((j >= 1) & (j <= n_chunks - NBUF))
            def _():
                pltpu.make_async_copy(
                    rows_v.at[b], out_hbm.at[pl.ds(out_base, CHUNK)], ssem
                ).wait()
                g = j - 1 + NBUF
                gather(g, lax.rem(g, NBUF))

            return carry

        lax.fori_loop(0, n_chunks, body, 0)

        for _ in range(NBUF):
            pltpu.make_async_copy(
                rows_v.at[0], out_hbm.at[pl.ds(out_base, CHUNK)], ssem
            ).wait()

    return k(idx3d, weight)


def kernel(idx, weight):
    b, h = idx.shape
    v, d = weight.shape
    n = b * h
    n_chunks = n // (NW * CHUNK)
    idx3d = idx.reshape(NW, n_chunks, CHUNK)
    out = _gather(idx3d, weight, n_chunks, d)
    return out.reshape(b, h, d)
